# baseline (speedup 1.0000x reference)
import functools

import jax
import jax.numpy as jnp
from jax import lax
from jax.experimental import pallas as pl
from jax.experimental.pallas import tpu as pltpu
from jax.experimental.pallas import tpu_sc as plsc

N_CAT = 26
N_CONT = 13
EMB_DIM = 32
BATCH = 4096
VOCAB = 1000000

NUM_CORES = 2
NUM_SUBCORES = 16

PACK = 128 // EMB_DIM
PROWS = VOCAB // PACK
CHUNK = 256
NCHUNK = BATCH // CHUNK

_mesh = plsc.VectorSubcoreMesh(core_axis_name="c", subcore_axis_name="s")


@functools.partial(
    pl.kernel,
    mesh=_mesh,
    compiler_params=pltpu.CompilerParams(use_tc_tiling_on_sc=True),
    out_type=jax.ShapeDtypeStruct((N_CAT, BATCH, 128), jnp.float32),
    scratch_types=[
        pltpu.VMEM((BATCH,), jnp.int32),
        pltpu.VMEM((CHUNK, 128), jnp.float32),
        pltpu.VMEM((CHUNK, 128), jnp.float32),
        pltpu.SemaphoreType.DMA,
        pltpu.SemaphoreType.DMA,
        pltpu.SemaphoreType.DMA,
        pltpu.SemaphoreType.DMA,
    ],
)
def _embed(offs_hbm, table_hbm, out_hbm, off_v, rows_a, rows_b,
           gsem_a, gsem_b, osem_a, osem_b):
    wid = lax.axis_index("s") * NUM_CORES + lax.axis_index("c")

    @pl.when(wid < N_CAT)
    def _gather_col():
        c = wid
        pltpu.sync_copy(offs_hbm.at[c], off_v)

        def fetch(k, rows_v, gsem):
            pltpu.async_copy(
                table_hbm.at[off_v.at[pl.ds(k * CHUNK, CHUNK)]],
                rows_v,
                gsem,
            )

        def put(k, rows_v, osem):
            pltpu.async_copy(
                rows_v,
                out_hbm.at[c, pl.ds(k * CHUNK, CHUNK), :],
                osem,
            )

        def gwait(rows_v, gsem):
            pltpu.make_async_copy(
                table_hbm.at[pl.ds(0, CHUNK)], rows_v, gsem
            ).wait()

        def owait(rows_v, osem):
            pltpu.make_async_copy(
                table_hbm.at[pl.ds(0, CHUNK)], rows_v, osem
            ).wait()

        fetch(0, rows_a, gsem_a)
        fetch(1, rows_b, gsem_b)

        def pair_body(kk, _):
            k = kk * 2
            gwait(rows_a, gsem_a)
            put(k, rows_a, osem_a)

            @pl.when(k + 2 < NCHUNK)
            def _():
                owait(rows_a, osem_a)
                fetch(k + 2, rows_a, gsem_a)

            gwait(rows_b, gsem_b)
            put(k + 1, rows_b, osem_b)

            @pl.when(k + 3 < NCHUNK)
            def _():
                owait(rows_b, osem_b)
                fetch(k + 3, rows_b, gsem_b)

            return _

        lax.fori_loop(0, NCHUNK // 2, pair_body, None)
        owait(rows_a, osem_a)
        owait(rows_b, osem_b)


def kernel(input, table):
    idx = input[:, :N_CAT].astype(jnp.int32)
    v = idx.T
    offs = v >> 2
    table4 = table.reshape(PROWS, 128)
    fat = _embed(offs, table4)
    sub = (v & 3)[:, :, None]
    parts = [fat[:, :, s * EMB_DIM:(s + 1) * EMB_DIM] for s in range(PACK)]
    emb3 = parts[0]
    for s in range(1, PACK):
        emb3 = jnp.where(sub == s, parts[s], emb3)
    flat = emb3.transpose(1, 0, 2).reshape(BATCH, N_CAT * EMB_DIM)
    return jnp.concatenate([flat, input[:, N_CAT:]], axis=1)


# speedup vs baseline: 1.2003x; 1.2003x over previous
import functools

import jax
import jax.numpy as jnp
from jax import lax
from jax.experimental import pallas as pl
from jax.experimental.pallas import tpu as pltpu
from jax.experimental.pallas import tpu_sc as plsc

N_CAT = 26
N_CONT = 13
EMB_DIM = 32
BATCH = 4096

NUM_CORES = 2
NUM_SUBCORES = 16
NW = NUM_CORES * NUM_SUBCORES
TOT = BATCH * N_CAT
PER_W = TOT // NW
CHUNK = 128
K = PER_W // CHUNK

_mesh = plsc.VectorSubcoreMesh(core_axis_name="c", subcore_axis_name="s")


@functools.partial(
    pl.kernel,
    mesh=_mesh,
    compiler_params=pltpu.CompilerParams(use_tc_tiling_on_sc=False),
    out_type=jax.ShapeDtypeStruct((TOT, EMB_DIM), jnp.float32),
    scratch_types=[
        pltpu.VMEM((K, CHUNK), jnp.int32),
        pltpu.VMEM((PER_W, EMB_DIM), jnp.float32),
        pltpu.SemaphoreType.DMA,
    ],
)
def _gather_rows(idx_hbm, table_hbm, out_hbm, idx_v, rows_v, sem):
    wid = lax.axis_index("s") * NUM_CORES + lax.axis_index("c")
    pltpu.sync_copy(idx_hbm.at[wid], idx_v)
    copies = [
        pltpu.async_copy(
            table_hbm.at[idx_v.at[j]],
            rows_v.at[pl.ds(j * CHUNK, CHUNK)],
            sem,
        )
        for j in range(K)
    ]
    for cp in copies:
        cp.wait()
    pltpu.sync_copy(rows_v, out_hbm.at[pl.ds(wid * PER_W, PER_W)])


def kernel(input, table):
    idx = input[:, :N_CAT].astype(jnp.int32).reshape(NW, K, CHUNK)
    emb = _gather_rows(idx, table)
    flat = emb.reshape(BATCH, N_CAT * EMB_DIM)
    return jnp.concatenate([flat, input[:, N_CAT:]], axis=1)


# speedup vs baseline: 1.3556x; 1.1294x over previous
import functools

import jax
import jax.numpy as jnp
from jax import lax
from jax.experimental import pallas as pl
from jax.experimental.pallas import tpu as pltpu
from jax.experimental.pallas import tpu_sc as plsc

N_CAT = 26
N_CONT = 13
EMB_DIM = 32
BATCH = 4096

NUM_CORES = 2
NUM_SUBCORES = 16
NW = NUM_CORES * NUM_SUBCORES
TOT = BATCH * N_CAT
PER_W = TOT // NW
CHUNK = 128
K = PER_W // CHUNK

_mesh = plsc.VectorSubcoreMesh(core_axis_name="c", subcore_axis_name="s")


@functools.partial(
    pl.kernel,
    mesh=_mesh,
    compiler_params=pltpu.CompilerParams(use_tc_tiling_on_sc=False),
    out_type=jax.ShapeDtypeStruct((TOT, EMB_DIM), jnp.float32),
    scratch_types=[
        pltpu.VMEM((K, CHUNK), jnp.int32),
        pltpu.VMEM((PER_W, EMB_DIM), jnp.float32),
        pltpu.SemaphoreType.DMA,
    ],
)
def _gather_rows(idx_hbm, table_hbm, out_hbm, idx_v, rows_v, sem):
    wid = lax.axis_index("s") * NUM_CORES + lax.axis_index("c")
    pltpu.sync_copy(idx_hbm.at[wid], idx_v)
    copies = [
        pltpu.async_copy(
            table_hbm.at[idx_v.at[j]],
            rows_v.at[pl.ds(j * CHUNK, CHUNK)],
            sem,
        )
        for j in range(K)
    ]
    for cp in copies:
        cp.wait()
    pltpu.sync_copy(rows_v, out_hbm.at[pl.ds(wid * PER_W, PER_W)])


VOCAB = 1000000
TCOLS = 2048
TGRID = (VOCAB + TCOLS - 1) // TCOLS
PROWS = VOCAB // 4


def _transpose_block(t_ref, out_ref, scr_ref):
    scr_ref[...] = t_ref[...].T
    for s in range(4):
        out_ref[:, s * EMB_DIM:(s + 1) * EMB_DIM] = (
            scr_ref[pl.ds(s, TCOLS // 4, 4), :]
        )


_linearize = pl.pallas_call(
    _transpose_block,
    grid=(TGRID,),
    in_specs=[pl.BlockSpec((EMB_DIM, TCOLS), lambda g: (0, g))],
    out_specs=pl.BlockSpec((TCOLS // 4, 128), lambda g: (g, 0)),
    out_shape=jax.ShapeDtypeStruct((PROWS, 128), jnp.float32),
    scratch_shapes=[pltpu.VMEM((TCOLS, EMB_DIM), jnp.float32)],
)


def kernel(input, table):
    idx = input[:, :N_CAT].astype(jnp.int32).reshape(NW, K, CHUNK)
    packed = _linearize(table.T)
    emb = _gather_rows(idx, packed.reshape(VOCAB, EMB_DIM))
    flat = emb.reshape(BATCH, N_CAT * EMB_DIM)
    return jnp.concatenate([flat, input[:, N_CAT:]], axis=1)


# speedup vs baseline: 1.9126x; 1.4109x over previous
import functools

import jax
import jax.numpy as jnp
from jax import lax
from jax.experimental import pallas as pl
from jax.experimental.pallas import tpu as pltpu
from jax.experimental.pallas import tpu_sc as plsc

N_CAT = 26
N_CONT = 13
EMB_DIM = 32
BATCH = 4096

NUM_CORES = 2
NUM_SUBCORES = 16
NW = NUM_CORES * NUM_SUBCORES
TOT = BATCH * N_CAT
PER_W = TOT // NW
CHUNK = 128
K = PER_W // CHUNK

_mesh = plsc.VectorSubcoreMesh(core_axis_name="c", subcore_axis_name="s")


@functools.partial(
    pl.kernel,
    mesh=_mesh,
    compiler_params=pltpu.CompilerParams(use_tc_tiling_on_sc=False),
    out_type=jax.ShapeDtypeStruct((TOT, EMB_DIM), jnp.float32),
    scratch_types=[
        pltpu.VMEM((K, CHUNK), jnp.int32),
        pltpu.VMEM((PER_W, EMB_DIM), jnp.float32),
        pltpu.SemaphoreType.DMA,
    ],
)
def _gather_rows(idx_hbm, table_hbm, out_hbm, idx_v, rows_v, sem):
    wid = lax.axis_index("s") * NUM_CORES + lax.axis_index("c")
    pltpu.sync_copy(idx_hbm.at[wid], idx_v)
    copies = [
        pltpu.async_copy(
            table_hbm.at[idx_v.at[j]],
            rows_v.at[pl.ds(j * CHUNK, CHUNK)],
            sem,
        )
        for j in range(K)
    ]
    for cp in copies:
        cp.wait()
    pltpu.sync_copy(rows_v, out_hbm.at[pl.ds(wid * PER_W, PER_W)])


VOCAB = 1000000
TCOLS = 8192
TGRID = (VOCAB + TCOLS - 1) // TCOLS
PROWS = VOCAB // 4


def _transpose_block(t_ref, out_ref, scr_ref):
    scr_ref[...] = t_ref[...].T
    for s in range(4):
        out_ref[:, s * EMB_DIM:(s + 1) * EMB_DIM] = (
            scr_ref[pl.ds(s, TCOLS // 4, 4), :]
        )


_linearize = pl.pallas_call(
    _transpose_block,
    grid=(TGRID,),
    in_specs=[pl.BlockSpec((EMB_DIM, TCOLS), lambda g: (0, g))],
    out_specs=pl.BlockSpec((TCOLS // 4, 128), lambda g: (g, 0)),
    out_shape=jax.ShapeDtypeStruct((PROWS, 128), jnp.float32),
    scratch_shapes=[pltpu.VMEM((TCOLS, EMB_DIM), jnp.float32)],
)


def kernel(input, table):
    idx = input[:, :N_CAT].astype(jnp.int32).reshape(NW, K, CHUNK)
    packed = _linearize(table.T)
    emb = _gather_rows(idx, packed.reshape(VOCAB, EMB_DIM))
    flat = emb.reshape(BATCH, N_CAT * EMB_DIM)
    return jnp.concatenate([flat, input[:, N_CAT:]], axis=1)


# speedup vs baseline: 1.9442x; 1.0165x over previous
import functools

import jax
import jax.numpy as jnp
from jax import lax
from jax.experimental import pallas as pl
from jax.experimental.pallas import tpu as pltpu
from jax.experimental.pallas import tpu_sc as plsc

N_CAT = 26
N_CONT = 13
EMB_DIM = 32
BATCH = 4096

NUM_CORES = 2
NUM_SUBCORES = 16
NW = NUM_CORES * NUM_SUBCORES
TOT = BATCH * N_CAT
PER_W = TOT // NW
CHUNK = 128
K = PER_W // CHUNK

_mesh = plsc.VectorSubcoreMesh(core_axis_name="c", subcore_axis_name="s")


@functools.partial(
    pl.kernel,
    mesh=_mesh,
    compiler_params=pltpu.CompilerParams(use_tc_tiling_on_sc=False),
    out_type=jax.ShapeDtypeStruct((TOT, EMB_DIM), jnp.float32),
    scratch_types=[
        pltpu.VMEM((K, CHUNK), jnp.int32),
        pltpu.VMEM((PER_W, EMB_DIM), jnp.float32),
        pltpu.SemaphoreType.DMA,
    ],
)
def _gather_rows(idx_hbm, table_hbm, out_hbm, idx_v, rows_v, sem):
    wid = lax.axis_index("s") * NUM_CORES + lax.axis_index("c")
    pltpu.sync_copy(idx_hbm.at[wid], idx_v)
    copies = [
        pltpu.async_copy(
            table_hbm.at[idx_v.at[j]],
            rows_v.at[pl.ds(j * CHUNK, CHUNK)],
            sem,
        )
        for j in range(K)
    ]
    for cp in copies:
        cp.wait()
    pltpu.sync_copy(rows_v, out_hbm.at[pl.ds(wid * PER_W, PER_W)])


VOCAB = 1000000
TCOLS = 32768
TGRID = (VOCAB + TCOLS - 1) // TCOLS
PROWS = VOCAB // 4


def _transpose_block(t_ref, out_ref, scr_ref):
    scr_ref[...] = t_ref[...].T
    for s in range(4):
        out_ref[:, s * EMB_DIM:(s + 1) * EMB_DIM] = (
            scr_ref[pl.ds(s, TCOLS // 4, 4), :]
        )


_linearize = pl.pallas_call(
    _transpose_block,
    grid=(TGRID,),
    in_specs=[pl.BlockSpec((EMB_DIM, TCOLS), lambda g: (0, g))],
    out_specs=pl.BlockSpec((TCOLS // 4, 128), lambda g: (g, 0)),
    out_shape=jax.ShapeDtypeStruct((PROWS, 128), jnp.float32),
    scratch_shapes=[pltpu.VMEM((TCOLS, EMB_DIM), jnp.float32)],
)


def kernel(input, table):
    idx = input[:, :N_CAT].astype(jnp.int32).reshape(NW, K, CHUNK)
    packed = _linearize(table.T)
    emb = _gather_rows(idx, packed.reshape(VOCAB, EMB_DIM))
    flat = emb.reshape(BATCH, N_CAT * EMB_DIM)
    return jnp.concatenate([flat, input[:, N_CAT:]], axis=1)
